# baseline (speedup 1.0000x reference)
import functools

import jax
import jax.numpy as jnp
from jax import lax
from jax.experimental import pallas as pl
from jax.experimental.pallas import tpu as pltpu
from jax.experimental.pallas import tpu_sc as plsc

EMB_DIM = 32
B, T, W = 1024, 26, 20
WP = 24
NUM_CORES = 2
NUM_SUBCORES = 16
LANES = 16
NUM_WORKERS = NUM_CORES * NUM_SUBCORES
CHUNK = 128
N_SLOTS = T * (WP // 8) * (B // CHUNK) * 8
S_PER_W = N_SLOTS // NUM_WORKERS
SLOTS_PER_T = (WP // 8) * (B // CHUNK) * 8
ET = EMB_DIM // 8

_mesh = plsc.VectorSubcoreMesh(core_axis_name="c", subcore_axis_name="s")


@functools.partial(
    pl.kernel,
    mesh=_mesh,
    compiler_params=pltpu.CompilerParams(
        needs_layout_passes=False, use_tc_tiling_on_sc=False),
    out_type=[
        jax.ShapeDtypeStruct((T, W, ET, B // CHUNK, 8, CHUNK), jnp.float32),
        jax.ShapeDtypeStruct((N_SLOTS * CHUNK,), jnp.float32),
        jax.ShapeDtypeStruct((ET, 8, CHUNK), jnp.float32),
        jax.ShapeDtypeStruct((6 * CHUNK, EMB_DIM), jnp.float32),
    ],
    scratch_types=[
        pltpu.VMEM((S_PER_W * CHUNK,), jnp.int32),
        pltpu.VMEM((S_PER_W * CHUNK,), jnp.float32),
        pltpu.VMEM((6 * CHUNK, EMB_DIM), jnp.float32),
        pltpu.VMEM((6 * CHUNK, EMB_DIM), jnp.float32),
        pltpu.VMEM((EMB_DIM, 131), jnp.float32),
        pltpu.VMEM((EMB_DIM, 131), jnp.float32),
        pltpu.VMEM((ET, 8, CHUNK), jnp.float32),
        pltpu.SemaphoreType.DMA,
        pltpu.SemaphoreType.DMA,
        pltpu.SemaphoreType.DMA,
        pltpu.SemaphoreType.DMA,
    ],
)
def _emb_lookup(xv_hbm, table_hbm, out_hbm, mask_hbm, dump_hbm, dummy_hbm,
                idx_v, mask_v, buf_a, buf_b, tile_a, tile_b, drain_v,
                gsem_a, gsem_b, wsem_a, wsem_b):
    wid = lax.axis_index("s") * NUM_CORES + lax.axis_index("c")
    wbase = wid * S_PER_W
    pltpu.sync_copy(
        xv_hbm.at[pl.ds(wbase * CHUNK, S_PER_W * CHUNK)], idx_v)

    def issue_group(s0, buf, gsem):
        for j in range(6):
            pltpu.async_copy(
                table_hbm.at[idx_v.at[pl.ds((s0 + j) * CHUNK, CHUNK)]],
                buf.at[pl.ds(j * CHUNK, CHUNK)], gsem)

    def drain_group(buf, gsem):
        pltpu.make_async_copy(dummy_hbm, buf, gsem).wait()

    def drain_writes(wsem):
        pltpu.make_async_copy(dump_hbm, drain_v, wsem).wait()

    issue_group(0, buf_a, gsem_a)

    def slot_block(i, carry):
        issue_group(6 * (2 * i + 1), buf_b, gsem_b)
        drain_group(buf_a, gsem_a)

        @pl.when(i < 12)
        def _pf():
            issue_group(6 * (2 * i + 2), buf_a, gsem_a)

        drain_group(buf_b, gsem_b)
        return carry

    lax.fori_loop(0, 13, slot_block, 0)
    pltpu.sync_copy(
        mask_v, mask_hbm.at[pl.ds(wbase * CHUNK, S_PER_W * CHUNK)])


def kernel(x, table):
    xv = jnp.pad(x.astype(jnp.int32).reshape(-1), (0, N_SLOTS * CHUNK - B * T * W))

    out6, mask_m, _, _ = _emb_lookup(xv, table)
    emb = out6.transpose(3, 5, 0, 1, 2, 4).reshape(B, T, W, EMB_DIM)
    m6 = mask_m.reshape(T, WP // 8, B // CHUNK, 8, CHUNK)
    mask = (m6.transpose(2, 4, 0, 1, 3)
            .reshape(B, T, WP)[:, :, :W])
    return emb, mask


# speedup vs baseline: 2.5170x; 2.5170x over previous
import functools

import jax
import jax.numpy as jnp
from jax import lax
from jax.experimental import pallas as pl
from jax.experimental.pallas import tpu as pltpu
from jax.experimental.pallas import tpu_sc as plsc

EMB_DIM = 32
N_TOTAL = 1024 * 26 * 20
NUM_CORES = 2
NUM_SUBCORES = 16
LANES = 16
NUM_WORKERS = NUM_CORES * NUM_SUBCORES
N_PER_W = N_TOTAL // NUM_WORKERS
CHUNK = 128
SUP_CHUNKS = 5
SUPER = CHUNK * SUP_CHUNKS
N_SUPERS = N_PER_W // SUPER
N_GROUPS = SUPER // LANES

_mesh = plsc.VectorSubcoreMesh(core_axis_name="c", subcore_axis_name="s")


@functools.partial(
    pl.kernel,
    mesh=_mesh,
    compiler_params=pltpu.CompilerParams(
        needs_layout_passes=False, use_tc_tiling_on_sc=False),
    out_type=[
        jax.ShapeDtypeStruct((N_TOTAL, EMB_DIM), jnp.float32),
        jax.ShapeDtypeStruct((N_TOTAL,), jnp.float32),
    ],
    scratch_types=[
        pltpu.VMEM((N_PER_W,), jnp.int32),
        pltpu.VMEM((N_PER_W,), jnp.float32),
        pltpu.VMEM((SUPER, EMB_DIM), jnp.float32),
        pltpu.VMEM((SUPER, EMB_DIM), jnp.float32),
        pltpu.SemaphoreType.DMA,
        pltpu.SemaphoreType.DMA,
    ],
)
def _emb_lookup(x_hbm, table_hbm, out_hbm, mask_hbm,
                idx_v, mask_v, buf_a, buf_b, sem_a, sem_b):
    wid = lax.axis_index("s") * NUM_CORES + lax.axis_index("c")
    base = wid * N_PER_W
    pltpu.sync_copy(x_hbm.at[pl.ds(base, N_PER_W)], idx_v)

    def issue(s, buf, sem):
        for j in range(SUP_CHUNKS):
            idx_chunk = idx_v.at[pl.ds(s * SUPER + j * CHUNK, CHUNK)]
            pltpu.async_copy(table_hbm.at[idx_chunk],
                             buf.at[pl.ds(j * CHUNK, CHUNK)], sem)

    def drain(buf, sem):
        pltpu.make_async_copy(out_hbm.at[pl.ds(0, SUPER)], buf, sem).wait()

    def process_write(s, buf):
        off = s * SUPER

        def mask_body(g, min_carry):
            v = idx_v[pl.ds(off + g * LANES, LANES)]
            mask_v[pl.ds(off + g * LANES, LANES)] = jnp.where(
                v != 0, jnp.float32(1.0), jnp.float32(0.0))
            return jnp.minimum(min_carry, v)

        minv = lax.fori_loop(0, N_GROUPS, mask_body,
                             jnp.full((LANES,), 1, jnp.int32), unroll=4)

        @pl.when(jnp.min(minv) == 0)
        def _zero_pad_rows():
            def zero_body(g, carry):
                v = idx_v[pl.ds(off + g * LANES, LANES)]
                pad = v == 0
                rows = lax.iota(jnp.int32, LANES) + g * LANES
                zeros = jnp.zeros((LANES,), jnp.float32)
                for col in range(EMB_DIM):
                    cols = jnp.full((LANES,), col, jnp.int32)
                    plsc.store_scatter(buf, [rows, cols], zeros, mask=pad)
                return carry

            lax.fori_loop(0, N_GROUPS, zero_body, 0)

        pltpu.sync_copy(buf, out_hbm.at[pl.ds(base + off, SUPER)])

    issue(0, buf_a, sem_a)

    def super_pair(i, carry):
        s0 = 2 * i
        issue(s0 + 1, buf_b, sem_b)
        drain(buf_a, sem_a)
        process_write(s0, buf_a)

        @pl.when(i < N_SUPERS // 2 - 1)
        def _prefetch_a():
            issue(s0 + 2, buf_a, sem_a)

        drain(buf_b, sem_b)
        process_write(s0 + 1, buf_b)
        return carry

    lax.fori_loop(0, N_SUPERS // 2, super_pair, 0)
    pltpu.sync_copy(mask_v, mask_hbm.at[pl.ds(base, N_PER_W)])


def kernel(x, table):
    x_flat = x.reshape(-1).astype(jnp.int32)
    out, mask = _emb_lookup(x_flat, table)
    emb = out.reshape(x.shape + (EMB_DIM,))
    return emb, mask.reshape(x.shape)


# speedup vs baseline: 3.7726x; 1.4989x over previous
import functools

import jax
import jax.numpy as jnp
from jax import lax
from jax.experimental import pallas as pl
from jax.experimental.pallas import tpu as pltpu
from jax.experimental.pallas import tpu_sc as plsc

EMB_DIM = 32
B, T, W = 1024, 26, 20
WP = 24
NUM_CORES = 2
NUM_SUBCORES = 16
LANES = 16
NUM_WORKERS = NUM_CORES * NUM_SUBCORES
CHUNK = 128
N_SLOTS = T * (WP // 8) * (B // CHUNK) * 8
S_PER_W = N_SLOTS // NUM_WORKERS
SLOTS_PER_T = (WP // 8) * (B // CHUNK) * 8
ET = EMB_DIM // 8

_mesh = plsc.VectorSubcoreMesh(core_axis_name="c", subcore_axis_name="s")


@functools.partial(
    pl.kernel,
    mesh=_mesh,
    compiler_params=pltpu.CompilerParams(
        needs_layout_passes=False, use_tc_tiling_on_sc=False),
    out_type=[
        jax.ShapeDtypeStruct((T, W, ET, B // CHUNK, 8, CHUNK), jnp.float32),
        jax.ShapeDtypeStruct((N_SLOTS * CHUNK,), jnp.float32),
        jax.ShapeDtypeStruct((ET, 8, CHUNK), jnp.float32),
    ],
    scratch_types=[
        pltpu.VMEM((S_PER_W * CHUNK,), jnp.int32),
        pltpu.VMEM((S_PER_W * CHUNK,), jnp.float32),
        [pltpu.VMEM((CHUNK, EMB_DIM), jnp.float32) for _ in range(12)],
        pltpu.VMEM((EMB_DIM, 131), jnp.float32),
        pltpu.VMEM((EMB_DIM, 131), jnp.float32),
        pltpu.VMEM((ET, 8, CHUNK), jnp.float32),
        [pltpu.SemaphoreType.DMA for _ in range(12)],
        pltpu.SemaphoreType.DMA,
        pltpu.SemaphoreType.DMA,
    ],
)
def _emb_lookup(xv_hbm, table_hbm, out_hbm, mask_hbm, dump_hbm,
                idx_v, mask_v, rows_ring, tile_a, tile_b, drain_v,
                gsem_ring, wsem_a, wsem_b):
    wid = lax.axis_index("s") * NUM_CORES + lax.axis_index("c")
    wbase = wid * S_PER_W
    pltpu.sync_copy(
        xv_hbm.at[pl.ds(wbase * CHUNK, S_PER_W * CHUNK)], idx_v)

    def slot_valid(s):
        r = (wbase + s) % SLOTS_PER_T
        return (r // 64) * 8 + r % 8 < W

    def issue(s, rows, gsem):
        @pl.when(slot_valid(s))
        def _():
            pltpu.async_copy(
                table_hbm.at[idx_v.at[pl.ds(s * CHUNK, CHUNK)]], rows, gsem)

    def drain_gather(s, rows, gsem):
        @pl.when(slot_valid(s))
        def _():
            pltpu.make_async_copy(
                table_hbm.at[pl.ds(0, CHUNK)], rows, gsem).wait()

    def drain_writes(wsem):
        pltpu.make_async_copy(dump_hbm, drain_v, wsem).wait()

    def process(s, rows, tile, wsem, first):
        sg = wbase + s
        t = sg // SLOTS_PER_T
        r = sg % SLOTS_PER_T
        wt = r // 64
        bt = (r % 64) // 8
        w8 = r % 8
        w = wt * 8 + w8
        off = s * CHUNK

        def mask_body(g, min_carry):
            v = idx_v[pl.ds(off + g * LANES, LANES)]
            mask_v[pl.ds(off + g * LANES, LANES)] = jnp.where(
                v != 0, jnp.float32(1.0), jnp.float32(0.0))
            return jnp.minimum(min_carry, v)

        minv = lax.fori_loop(0, CHUNK // LANES, mask_body,
                             jnp.full((LANES,), 1, jnp.int32), unroll=8)

        @pl.when(jnp.min(minv) == 0)
        def _zero_pad_rows():
            zeros = jnp.zeros((LANES,), jnp.float32)

            def zero_body(g, carry):
                v = idx_v[pl.ds(off + g * LANES, LANES)]
                pad = v == 0
                rws = lax.iota(jnp.int32, LANES) + g * LANES
                for col in range(EMB_DIM):
                    cols = jnp.full((LANES,), col, jnp.int32)
                    plsc.store_scatter(rows, [rws, cols], zeros, mask=pad)
                return carry

            lax.fori_loop(0, CHUNK // LANES, zero_body, 0)

        @pl.when(jnp.logical_not(first))
        def _drain_prev():
            drain_writes(wsem)

        iota16 = lax.iota(jnp.int32, LANES)
        evec0 = iota16
        evec1 = iota16 + LANES

        def transpose_body(b, carry):
            colv = jnp.full((LANES,), 0, jnp.int32) + b
            plsc.store_scatter(tile, [evec0, colv], rows[b, pl.ds(0, LANES)])
            plsc.store_scatter(tile, [evec1, colv],
                               rows[b, pl.ds(LANES, LANES)])
            return carry

        lax.fori_loop(0, CHUNK, transpose_body, 0, unroll=4)

        @pl.when(w < W)
        def _write_real():
            for et in range(ET):
                pltpu.async_copy(tile.at[pl.ds(et * 8, 8), pl.ds(0, CHUNK)],
                                 out_hbm.at[t, w, et, bt], wsem)

        @pl.when(w >= W)
        def _write_dump():
            for et in range(ET):
                pltpu.async_copy(tile.at[pl.ds(et * 8, 8), pl.ds(0, CHUNK)],
                                 dump_hbm.at[et], wsem)

    NBUF = len(rows_ring)
    for b in range(NBUF):
        issue(b, rows_ring[b], gsem_ring[b])

    def slot_block(i, carry):
        for b in range(NBUF):
            s = i * NBUF + b
            drain_gather(s, rows_ring[b], gsem_ring[b])
            tile, wsem = (tile_a, wsem_a) if b % 2 == 0 else (tile_b, wsem_b)
            process(s, rows_ring[b], tile, wsem,
                    jnp.logical_and(i == 0, b // 2 == 0))

            @pl.when(i < S_PER_W // NBUF - 1)
            def _prefetch():
                issue(s + NBUF, rows_ring[b], gsem_ring[b])

        return carry

    lax.fori_loop(0, S_PER_W // NBUF, slot_block, 0)
    drain_writes(wsem_a)
    drain_writes(wsem_b)
    pltpu.sync_copy(
        mask_v, mask_hbm.at[pl.ds(wbase * CHUNK, S_PER_W * CHUNK)])


def kernel(x, table):
    px = jnp.pad(x.astype(jnp.int32), ((0, 0), (0, 0), (0, WP - W)))
    v = px.transpose(1, 2, 0).reshape(T, WP // 8, 8, B // CHUNK, CHUNK)
    xv = v.transpose(0, 1, 3, 2, 4).reshape(N_SLOTS * CHUNK)

    out6, mask_m, _ = _emb_lookup(xv, table)
    emb = out6.transpose(3, 5, 0, 1, 2, 4).reshape(B, T, W, EMB_DIM)
    m6 = mask_m.reshape(T, WP // 8, B // CHUNK, 8, CHUNK)
    mask = (m6.transpose(2, 4, 0, 1, 3)
            .reshape(B, T, WP)[:, :, :W])
    return emb, mask
